# baseline (speedup 1.0000x reference)
import jax
import jax.numpy as jnp
from jax import lax
from jax.experimental import pallas as pl
from jax.experimental.pallas import tpu as pltpu
from jax.experimental.pallas import tpu_sc as plsc

NC = 2
NS = 16
L = 16
NW = NC * NS
N_BATCH = 16384
N_FACT = 32
BPW = N_BATCH // NW
CHUNK = 128
NCHUNK = BPW // CHUNK


def _dot_body(aid1_hbm, aid2_hbm, table_hbm, out_hbm,
              idx1_v, idx2_v, rows1_v, rows2_v, out_v, sem):
    wid = lax.axis_index("s") * NC + lax.axis_index("c")

    pltpu.sync_copy(aid1_hbm.at[pl.ds(wid * NCHUNK, NCHUNK)], idx1_v)
    pltpu.sync_copy(aid2_hbm.at[pl.ds(wid * NCHUNK, NCHUNK)], idx2_v)

    copies = []
    for j in range(NCHUNK):
        copies.append(pltpu.async_copy(
            table_hbm.at[idx1_v.at[j]],
            rows1_v.at[pl.ds(j * CHUNK, CHUNK)], sem))
        copies.append(pltpu.async_copy(
            table_hbm.at[idx2_v.at[j]],
            rows2_v.at[pl.ds(j * CHUNK, CHUNK)], sem))
    for c in copies:
        c.wait()

    lane = lax.iota(jnp.int32, L)

    def group(g, carry):
        row = g * L + lane
        acc = jnp.zeros((L,), jnp.float32)
        for d in range(N_FACT):
            col = jnp.full((L,), d, jnp.int32)
            a = plsc.load_gather(rows1_v, [row, col])
            b = plsc.load_gather(rows2_v, [row, col])
            acc = acc + a * b
        out_v[pl.ds(g * L, L)] = acc
        return carry

    lax.fori_loop(0, BPW // L, group, 0)

    pltpu.sync_copy(out_v, out_hbm.at[pl.ds(wid * BPW, BPW)])


def kernel(aid1, aid2, table):
    a1 = aid1.astype(jnp.int32).reshape(NW * NCHUNK, CHUNK)
    a2 = aid2.astype(jnp.int32).reshape(NW * NCHUNK, CHUNK)
    mesh = plsc.VectorSubcoreMesh(core_axis_name="c", subcore_axis_name="s")
    f = pl.kernel(
        _dot_body,
        out_type=jax.ShapeDtypeStruct((N_BATCH,), jnp.float32),
        mesh=mesh,
        scratch_types=[
            pltpu.VMEM((NCHUNK, CHUNK), jnp.int32),
            pltpu.VMEM((NCHUNK, CHUNK), jnp.int32),
            pltpu.VMEM((BPW, N_FACT), jnp.float32),
            pltpu.VMEM((BPW, N_FACT), jnp.float32),
            pltpu.VMEM((BPW,), jnp.float32),
            pltpu.SemaphoreType.DMA,
        ],
        compiler_params=pltpu.CompilerParams(
            needs_layout_passes=False, use_tc_tiling_on_sc=False),
    )
    return f(a1, a2, table)


# speedup vs baseline: 1.0024x; 1.0024x over previous
import jax
import jax.numpy as jnp
from jax import lax
from jax.experimental import pallas as pl
from jax.experimental.pallas import tpu as pltpu
from jax.experimental.pallas import tpu_sc as plsc

NC = 2
NS = 16
L = 16
NW = NC * NS
N_BATCH = 16384
N_FACT = 32
PADF = 128
BPW = N_BATCH // NW
CHUNK = 128


def _dot_body(aids_hbm, tpad_hbm, out_hbm, idx_v, buf1_v, buf2_v, out_v, sem):
    wid = lax.axis_index("s") * NC + lax.axis_index("c")

    pltpu.sync_copy(aids_hbm.at[wid], idx_v)

    lane = lax.iota(jnp.int32, L)

    for r in range(4):
        c1 = pltpu.async_copy(tpad_hbm.at[idx_v.at[r]], buf1_v, sem)
        c2 = pltpu.async_copy(tpad_hbm.at[idx_v.at[r + 4]], buf2_v, sem)
        c1.wait()
        c2.wait()
        for s in range(CHUNK // L):
            row = s * L + lane
            acc = jnp.zeros((L,), jnp.float32)
            for f in range(N_FACT):
                col = jnp.full((L,), f, jnp.int32)
                a = plsc.load_gather(buf1_v, [row, col])
                b = plsc.load_gather(buf2_v, [row, col])
                acc = acc + a * b
            out_v[pl.ds(r * CHUNK + s * L, L)] = acc

    pltpu.sync_copy(out_v, out_hbm.at[pl.ds(wid * BPW, BPW)])


def kernel(aid1, aid2, table):
    a1 = aid1.astype(jnp.int32).reshape(NW, 4, 128)
    a2 = aid2.astype(jnp.int32).reshape(NW, 4, 128)
    aids = jnp.concatenate([a1, a2], axis=1)
    tpad = jnp.pad(table, ((0, 0), (0, PADF - N_FACT)))
    mesh = plsc.VectorSubcoreMesh(core_axis_name="c", subcore_axis_name="s")
    f = pl.kernel(
        _dot_body,
        out_type=jax.ShapeDtypeStruct((N_BATCH,), jnp.float32),
        mesh=mesh,
        scratch_types=[
            pltpu.VMEM((8, 128), jnp.int32),
            pltpu.VMEM((CHUNK, PADF), jnp.float32),
            pltpu.VMEM((CHUNK, PADF), jnp.float32),
            pltpu.VMEM((BPW,), jnp.float32),
            pltpu.SemaphoreType.DMA,
        ],
        compiler_params=pltpu.CompilerParams(
            needs_layout_passes=False, use_tc_tiling_on_sc=True),
    )
    return f(aids, tpad)
